# baseline (speedup 1.0000x reference)
import functools

import jax
import jax.numpy as jnp
from jax.experimental import pallas as pl
from jax.experimental.pallas import tpu as pltpu


def _mega_kernel(y_ref, ei_ref, w_ref, b_ref, o_ref, y_s, y2_s, a_s,
                 *, n, d, e, bi):
    t = pl.program_id(0)

    @pl.when(t == 0)
    def _build_a():
        acc = jnp.zeros((n, n), jnp.float32)
        chunk = 2048
        for c in range(0, e, chunk):
            src = ei_ref[0:1, c:c + chunk]
            dst = ei_ref[1:2, c:c + chunk]
            iota = jax.lax.broadcasted_iota(jnp.int32, (n, chunk), 0)
            s_oh = (iota == src).astype(jnp.float32)
            d_oh = (iota == dst).astype(jnp.float32)
            acc += jax.lax.dot_general(
                s_oh, d_oh, (((1,), (1,)), ((), ())),
                preferred_element_type=jnp.float32)
        a_s[...] = acc

    @pl.when(t < 8)
    def _phase_1():
        y_s[t] = y_ref[...]
        a = a_s[...].astype(jnp.bfloat16)
        for dd in range(8):
            ys = y_ref[:, dd, :]
            y2_s[t, :, dd, :] = jax.lax.dot_general(
                a, ys, (((0,), (0,)), ((), ())),
                preferred_element_type=jnp.float32)

    @pl.when(t >= 8)
    def _phase_2():
        m = t - 8
        ys = [y_s[:, m * bi + i, :, :].reshape(d, n) for i in range(bi)]
        y2s = [y2_s[:, m * bi + i, :, :].reshape(d, n) for i in range(bi)]
        y_flat = jnp.concatenate(ys, axis=0)
        y1_flat = jax.lax.dot_general(
            y_flat, a_s[...].astype(jnp.bfloat16), (((1,), (0,)), ((), ())),
            preferred_element_type=jnp.float32).astype(jnp.bfloat16)
        cat = jnp.concatenate(
            [jnp.concatenate(
                [ys[i], y1_flat[i * d:(i + 1) * d, :],
                 y2s[i].astype(jnp.bfloat16)], axis=0)
             for i in range(bi)], axis=1)
        res = jax.lax.dot_general(
            cat, w_ref[...].astype(jnp.bfloat16), (((0,), (0,)), ((), ())),
            preferred_element_type=jnp.float32)
        o_ref[...] = jnp.maximum(res + b_ref[...], 0.0)


def kernel(X, edge_index, W, b):
    n, n2, d = X.shape
    assert n == n2 and d % 8 == 0 and n % 8 == 0
    e = edge_index.shape[1]
    b2 = b.reshape(1, d)
    bi = n // 8

    y = jnp.transpose(X, (0, 2, 1)).astype(jnp.bfloat16)

    out_flat = pl.pallas_call(
        functools.partial(_mega_kernel, n=n, d=d, e=e, bi=bi),
        grid=(16,),
        in_specs=[
            pl.BlockSpec((n, 8, n), lambda t: (0, jnp.minimum(t, 7), 0)),
            pl.BlockSpec((2, e), lambda t: (0, 0)),
            pl.BlockSpec((3 * d, d), lambda t: (0, 0)),
            pl.BlockSpec((1, d), lambda t: (0, 0)),
        ],
        out_specs=pl.BlockSpec(
            (bi * n, d), lambda t: (jnp.maximum(t - 8, 0), 0)),
        out_shape=jax.ShapeDtypeStruct((n * n, d), jnp.float32),
        scratch_shapes=[
            pltpu.VMEM((d // 8, n, 8, n), jnp.bfloat16),
            pltpu.VMEM((d // 8, n, 8, n), jnp.float32),
            pltpu.VMEM((n, n), jnp.float32),
        ],
    )(y, edge_index, W, b2)

    return out_flat.reshape(n, n, d)


# speedup vs baseline: 1.2351x; 1.2351x over previous
import functools

import jax
import jax.numpy as jnp
from jax.experimental import pallas as pl
from jax.experimental.pallas import tpu as pltpu


def _mega_kernel(y_ref, ei_ref, w_ref, b_ref, o_ref, y_s, y2_s, a_s,
                 *, n, d, e, bi):
    t = pl.program_id(0)

    @pl.when(t == 0)
    def _build_a():
        acc = jnp.zeros((n, n), jnp.float32)
        chunk = 2048
        for c in range(0, e, chunk):
            src = ei_ref[0:1, c:c + chunk]
            dst = ei_ref[1:2, c:c + chunk]
            iota = jax.lax.broadcasted_iota(jnp.int32, (n, chunk), 0)
            s_oh = (iota == src).astype(jnp.float32)
            d_oh = (iota == dst).astype(jnp.float32)
            acc += jax.lax.dot_general(
                s_oh, d_oh, (((1,), (1,)), ((), ())),
                preferred_element_type=jnp.float32)
        a_s[...] = acc

    @pl.when(t < 8)
    def _phase_1():
        y_s[t] = y_ref[...]
        a = a_s[...]
        for dd in range(8):
            ys = y_ref[:, dd, :]
            y2_s[t, :, dd, :] = jax.lax.dot_general(
                a, ys, (((0,), (0,)), ((), ())),
                preferred_element_type=jnp.float32)

    @pl.when(t >= 8)
    def _phase_2():
        m = t - 8
        ys = [y_s[:, m * bi + i, :, :].reshape(d, n) for i in range(bi)]
        y2s = [y2_s[:, m * bi + i, :, :].reshape(d, n) for i in range(bi)]
        y_flat = jnp.concatenate(ys, axis=0)
        y1_flat = jax.lax.dot_general(
            y_flat, a_s[...], (((1,), (0,)), ((), ())),
            preferred_element_type=jnp.float32)
        cat = jnp.concatenate(
            [jnp.concatenate(
                [ys[i], y1_flat[i * d:(i + 1) * d, :], y2s[i]], axis=0)
             for i in range(bi)], axis=1)
        res = jax.lax.dot_general(
            cat, w_ref[...], (((0,), (0,)), ((), ())),
            preferred_element_type=jnp.float32)
        o_ref[...] = jnp.maximum(res + b_ref[...], 0.0)


def kernel(X, edge_index, W, b):
    n, n2, d = X.shape
    assert n == n2 and d % 8 == 0 and n % 8 == 0
    e = edge_index.shape[1]
    b2 = b.reshape(1, d)
    bi = n // 8

    y = jnp.transpose(X, (0, 2, 1))

    out_flat = pl.pallas_call(
        functools.partial(_mega_kernel, n=n, d=d, e=e, bi=bi),
        grid=(16,),
        in_specs=[
            pl.BlockSpec((n, 8, n), lambda t: (0, jnp.minimum(t, 7), 0)),
            pl.BlockSpec((2, e), lambda t: (0, 0)),
            pl.BlockSpec((3 * d, d), lambda t: (0, 0)),
            pl.BlockSpec((1, d), lambda t: (0, 0)),
        ],
        out_specs=pl.BlockSpec(
            (bi * n, d), lambda t: (jnp.maximum(t - 8, 0), 0)),
        out_shape=jax.ShapeDtypeStruct((n * n, d), jnp.float32),
        scratch_shapes=[
            pltpu.VMEM((d // 8, n, 8, n), jnp.float32),
            pltpu.VMEM((d // 8, n, 8, n), jnp.float32),
            pltpu.VMEM((n, n), jnp.float32),
        ],
    )(y, edge_index, W, b2)

    return out_flat.reshape(n, n, d)
